# baseline (speedup 1.0000x reference)
import dataclasses
import functools

import jax
import jax.numpy as jnp
from jax import lax
from jax.experimental import pallas as pl
from jax.experimental.pallas import tpu as pltpu
from jax.experimental.pallas import tpu_sc as plsc

B, N, S = 8, 4096, 1024
C1, C2 = 256, 512
O1, O2 = 256, 256
EPS_BN = 1e-5
NB = 512
NBLK = N // NB
CNT = float(B * N)
M = B * N
NWORK = 32
PPW = M // NWORK
WIN = 64
NWIN = PPW // WIN


def _g_kernel(f2_ref, w1b_ref, g_ref):
    g_ref[0] = jax.lax.dot_general(
        f2_ref[0], w1b_ref[...], (((1,), (1,)), ((), ())),
        preferred_element_type=jnp.float32)


def _knn_kernel(p1_ref, p2_ref, idx_ref, w_ref):
    p1 = p1_ref[0]
    p2 = p2_ref[0]
    d = jnp.zeros((NB, S), jnp.float32)
    for j in range(3):
        t = p1[:, j][:, None] - p2[:, j][None, :]
        d = d + t * t
    iota = jax.lax.broadcasted_iota(jnp.int32, (NB, S), 1)
    key = jax.lax.bitcast_convert_type(
        (jax.lax.bitcast_convert_type(d, jnp.int32) & -1024) | iota,
        jnp.float32)
    m1 = jnp.full((NB, 128), 3.0e38, jnp.float32)
    m2 = m1
    m3 = m1
    for c in range(8):
        x = key[:, c * 128:(c + 1) * 128]
        hi = jnp.maximum(m1, x)
        m1 = jnp.minimum(m1, x)
        hi2 = jnp.maximum(m2, hi)
        m2 = jnp.minimum(m2, hi)
        m3 = jnp.minimum(m3, hi2)
    ks = []
    for r in range(3):
        k = jnp.min(m1, axis=1, keepdims=True)
        ks.append(k)
        if r < 2:
            sel = (m1 == k)
            m1 = jnp.where(sel, m2, m1)
            m2 = jnp.where(sel, m3, m2)
    recips = []
    gidxs = []
    boff = pl.program_id(0) * S
    for k in ks:
        ki = jax.lax.bitcast_convert_type(k, jnp.int32)
        dk = jax.lax.bitcast_convert_type(ki & -1024, jnp.float32)
        recips.append(1.0 / (dk + 1e-8))
        gidxs.append((ki & 1023) + boff)
    wsum = recips[0] + recips[1] + recips[2]
    idx_ref[0] = jnp.concatenate(
        gidxs + [jnp.zeros((NB, 1), jnp.int32)], axis=1)
    w_ref[0] = jnp.concatenate(
        [recips[0] / wsum, recips[1] / wsum, recips[2] / wsum,
         jnp.zeros((NB, 1), jnp.float32)], axis=1)


def _sc_interp(g2, i0, i1, i2, w0, w1, w2, out,
               i0v, i1v, i2v, w0v, w1v, w2v, r0, r1, r2, ov, sem):
    wid = lax.axis_index("s") * 2 + lax.axis_index("c")

    @pl.loop(0, NWIN)
    def _(wi):
        base = wid * PPW + wi * WIN
        sl = pl.ds(base, WIN)
        pltpu.sync_copy(i0.at[sl], i0v)
        pltpu.sync_copy(i1.at[sl], i1v)
        pltpu.sync_copy(i2.at[sl], i2v)
        pltpu.sync_copy(w0.at[sl], w0v)
        pltpu.sync_copy(w1.at[sl], w1v)
        pltpu.sync_copy(w2.at[sl], w2v)
        c0 = pltpu.async_copy(g2.at[i0v], r0, sem)
        c1 = pltpu.async_copy(g2.at[i1v], r1, sem)
        c2 = pltpu.async_copy(g2.at[i2v], r2, sem)
        c0.wait()
        c1.wait()
        c2.wait()

        @pl.loop(0, WIN)
        def _(i):
            iv = jnp.broadcast_to(i, (16,))
            s0 = plsc.load_gather(w0v, [iv])
            s1 = plsc.load_gather(w1v, [iv])
            s2 = plsc.load_gather(w2v, [iv])
            for c in range(O1 // 16):
                cs = pl.ds(c * 16, 16)
                ov[i, cs] = (s0 * r0[i, cs] + s1 * r1[i, cs]
                             + s2 * r2[i, cs])

        pltpu.sync_copy(ov, out.at[sl])


def _combine_kernel(i_ref, f1_ref, w1a_ref, h1_ref, st_ref):
    h1 = i_ref[0] + jax.lax.dot_general(
        f1_ref[0], w1a_ref[...], (((1,), (1,)), ((), ())),
        preferred_element_type=jnp.float32)
    h1_ref[0] = h1
    part = jnp.concatenate([jnp.sum(h1, axis=0)[None, :],
                            jnp.sum(h1 * h1, axis=0)[None, :]], axis=0)
    first = (pl.program_id(0) == 0) & (pl.program_id(1) == 0)

    @pl.when(first)
    def _():
        st_ref[...] = part

    @pl.when(jnp.logical_not(first))
    def _():
        st_ref[...] += part


def _bn_affine_in_kernel(st_ref, g_ref, b_ref):
    mean = st_ref[0:1, :] * (1.0 / CNT)
    var = jnp.maximum(st_ref[1:2, :] * (1.0 / CNT) - mean * mean, 0.0)
    scale = g_ref[...] * jax.lax.rsqrt(var + EPS_BN)
    shift = b_ref[...] - mean * scale
    return scale, shift


def _layer2_kernel(h1_ref, st1_ref, g_ref, b_ref, w2_ref, h2_ref, st_ref):
    scale, shift = _bn_affine_in_kernel(st1_ref, g_ref, b_ref)
    x = jnp.maximum(h1_ref[0] * scale + shift, 0.0)
    h2 = jax.lax.dot_general(
        x, w2_ref[...], (((1,), (1,)), ((), ())),
        preferred_element_type=jnp.float32)
    h2_ref[0] = h2
    part = jnp.concatenate([jnp.sum(h2, axis=0)[None, :],
                            jnp.sum(h2 * h2, axis=0)[None, :]], axis=0)
    first = (pl.program_id(0) == 0) & (pl.program_id(1) == 0)

    @pl.when(first)
    def _():
        st_ref[...] = part

    @pl.when(jnp.logical_not(first))
    def _():
        st_ref[...] += part


def _final_kernel(h2_ref, st2_ref, g_ref, b_ref, o_ref):
    scale, shift = _bn_affine_in_kernel(st2_ref, g_ref, b_ref)
    o_ref[0] = jnp.maximum(h2_ref[0] * scale + shift, 0.0)


@jax.jit
def _run(point_1, point_2, point_feat_1, point_feat_2, W1, g1, b1, W2, g2, b2):
    W1a = W1[:, :C1]
    W1b = W1[:, C1:]
    g1r = g1.reshape(1, O1)
    b1r = b1.reshape(1, O1)
    g2r = g2.reshape(1, O2)
    b2r = b2.reshape(1, O2)

    G = pl.pallas_call(
        _g_kernel,
        grid=(B,),
        in_specs=[
            pl.BlockSpec((1, S, C2), lambda b: (b, 0, 0)),
            pl.BlockSpec((O1, C2), lambda b: (0, 0)),
        ],
        out_specs=pl.BlockSpec((1, S, O1), lambda b: (b, 0, 0)),
        out_shape=jax.ShapeDtypeStruct((B, S, O1), jnp.float32),
    )(point_feat_2, W1b)

    idx4, w4 = pl.pallas_call(
        _knn_kernel,
        grid=(B, NBLK),
        in_specs=[
            pl.BlockSpec((1, NB, 3), lambda b, i: (b, i, 0)),
            pl.BlockSpec((1, S, 3), lambda b, i: (b, 0, 0)),
        ],
        out_specs=[
            pl.BlockSpec((1, NB, 4), lambda b, i: (b, i, 0)),
            pl.BlockSpec((1, NB, 4), lambda b, i: (b, i, 0)),
        ],
        out_shape=[
            jax.ShapeDtypeStruct((B, N, 4), jnp.int32),
            jax.ShapeDtypeStruct((B, N, 4), jnp.float32),
        ],
    )(point_1, point_2)

    idxf = idx4.reshape(M, 4)
    wf = w4.reshape(M, 4)
    g2f = G.reshape(B * S, O1)

    cp = pltpu.CompilerParams()
    if "needs_layout_passes" in pltpu.CompilerParams.__dataclass_fields__:
        cp = dataclasses.replace(cp, needs_layout_passes=False)
    sc_interp = functools.partial(
        pl.kernel,
        out_type=jax.ShapeDtypeStruct((M, O1), jnp.float32),
        compiler_params=cp,
        mesh=plsc.VectorSubcoreMesh(core_axis_name="c", subcore_axis_name="s"),
        scratch_types=[
            pltpu.VMEM((WIN,), jnp.int32),
            pltpu.VMEM((WIN,), jnp.int32),
            pltpu.VMEM((WIN,), jnp.int32),
            pltpu.VMEM((WIN,), jnp.float32),
            pltpu.VMEM((WIN,), jnp.float32),
            pltpu.VMEM((WIN,), jnp.float32),
            pltpu.VMEM((WIN, O1), jnp.float32),
            pltpu.VMEM((WIN, O1), jnp.float32),
            pltpu.VMEM((WIN, O1), jnp.float32),
            pltpu.VMEM((WIN, O1), jnp.float32),
            pltpu.SemaphoreType.DMA,
        ],
    )(_sc_interp)

    interp = sc_interp(g2f, idxf[:, 0], idxf[:, 1], idxf[:, 2],
                       wf[:, 0], wf[:, 1], wf[:, 2])
    interp = interp.reshape(B, N, O1)

    h1, st1 = pl.pallas_call(
        _combine_kernel,
        grid=(B, NBLK),
        in_specs=[
            pl.BlockSpec((1, NB, O1), lambda b, i: (b, i, 0)),
            pl.BlockSpec((1, NB, C1), lambda b, i: (b, i, 0)),
            pl.BlockSpec((O1, C1), lambda b, i: (0, 0)),
        ],
        out_specs=[
            pl.BlockSpec((1, NB, O1), lambda b, i: (b, i, 0)),
            pl.BlockSpec((2, O1), lambda b, i: (0, 0)),
        ],
        out_shape=[
            jax.ShapeDtypeStruct((B, N, O1), jnp.float32),
            jax.ShapeDtypeStruct((2, O1), jnp.float32),
        ],
    )(interp, point_feat_1, W1a)

    h2, st2 = pl.pallas_call(
        _layer2_kernel,
        grid=(B, NBLK),
        in_specs=[
            pl.BlockSpec((1, NB, O1), lambda b, i: (b, i, 0)),
            pl.BlockSpec((2, O1), lambda b, i: (0, 0)),
            pl.BlockSpec((1, O1), lambda b, i: (0, 0)),
            pl.BlockSpec((1, O1), lambda b, i: (0, 0)),
            pl.BlockSpec((O2, O1), lambda b, i: (0, 0)),
        ],
        out_specs=[
            pl.BlockSpec((1, NB, O2), lambda b, i: (b, i, 0)),
            pl.BlockSpec((2, O2), lambda b, i: (0, 0)),
        ],
        out_shape=[
            jax.ShapeDtypeStruct((B, N, O2), jnp.float32),
            jax.ShapeDtypeStruct((2, O2), jnp.float32),
        ],
    )(h1, st1, g1r, b1r, W2)

    out = pl.pallas_call(
        _final_kernel,
        grid=(B, NBLK),
        in_specs=[
            pl.BlockSpec((1, NB, O2), lambda b, i: (b, i, 0)),
            pl.BlockSpec((2, O2), lambda b, i: (0, 0)),
            pl.BlockSpec((1, O2), lambda b, i: (0, 0)),
            pl.BlockSpec((1, O2), lambda b, i: (0, 0)),
        ],
        out_specs=pl.BlockSpec((1, NB, O2), lambda b, i: (b, i, 0)),
        out_shape=jax.ShapeDtypeStruct((B, N, O2), jnp.float32),
    )(h2, st2, g2r, b2r)

    return out


def kernel(point_1, point_2, point_feat_1, point_feat_2, W1, g1, b1, W2, g2, b2):
    return _run(point_1, point_2, point_feat_1, point_feat_2,
                W1, g1, b1, W2, g2, b2)


# speedup vs baseline: 1.4799x; 1.4799x over previous
import dataclasses
import functools

import jax
import jax.numpy as jnp
from jax import lax
from jax.experimental import pallas as pl
from jax.experimental.pallas import tpu as pltpu
from jax.experimental.pallas import tpu_sc as plsc

B, N, S = 8, 4096, 1024
C1, C2 = 256, 512
O1, O2 = 256, 256
EPS_BN = 1e-5
NB = 512
NBLK = N // NB
NB2 = 2048
NBLK2 = N // NB2
CNT = float(B * N)
M = B * N
NWORK = 32
PPW = M // NWORK
WIN = 64
NWIN = PPW // WIN


def _g_kernel(f2_ref, w1b_ref, g_ref):
    g_ref[0] = jax.lax.dot_general(
        f2_ref[0], w1b_ref[...], (((1,), (1,)), ((), ())),
        preferred_element_type=jnp.float32)


def _knn_kernel(p1_ref, p2_ref, idx_ref, w_ref):
    p1 = p1_ref[0]
    p2 = p2_ref[0]
    d = jnp.zeros((NB, S), jnp.float32)
    for j in range(3):
        t = p1[:, j][:, None] - p2[:, j][None, :]
        d = d + t * t
    iota = jax.lax.broadcasted_iota(jnp.int32, (NB, S), 1)
    key = jax.lax.bitcast_convert_type(
        (jax.lax.bitcast_convert_type(d, jnp.int32) & -1024) | iota,
        jnp.float32)
    m1 = jnp.full((NB, 128), 3.0e38, jnp.float32)
    m2 = m1
    m3 = m1
    for c in range(8):
        x = key[:, c * 128:(c + 1) * 128]
        hi = jnp.maximum(m1, x)
        m1 = jnp.minimum(m1, x)
        hi2 = jnp.maximum(m2, hi)
        m2 = jnp.minimum(m2, hi)
        m3 = jnp.minimum(m3, hi2)
    ks = []
    for r in range(3):
        k = jnp.min(m1, axis=1, keepdims=True)
        ks.append(k)
        if r < 2:
            sel = (m1 == k)
            m1 = jnp.where(sel, m2, m1)
            m2 = jnp.where(sel, m3, m2)
    recips = []
    gidxs = []
    boff = pl.program_id(0) * S
    for k in ks:
        ki = jax.lax.bitcast_convert_type(k, jnp.int32)
        dk = jax.lax.bitcast_convert_type(ki & -1024, jnp.float32)
        recips.append(1.0 / (dk + 1e-8))
        gidxs.append((ki & 1023) + boff)
    wsum = recips[0] + recips[1] + recips[2]
    idx_ref[0] = jnp.concatenate(
        gidxs + [jnp.zeros((NB, 1), jnp.int32)], axis=1)
    w_ref[0] = jnp.concatenate(
        [recips[0] / wsum, recips[1] / wsum, recips[2] / wsum,
         jnp.zeros((NB, 1), jnp.float32)], axis=1)


def _sc_interp(g2, iflat, wflat, out,
               iallv, wallv, i0v, i1v, i2v, r0, r1, r2, ov, sem):
    wid = lax.axis_index("s") * 2 + lax.axis_index("c")
    iota16 = jax.lax.iota(jnp.int32, 16)
    ivs = (i0v, i1v, i2v)

    @pl.loop(0, NWIN)
    def _(wi):
        base = wid * PPW + wi * WIN
        pltpu.sync_copy(iflat.at[pl.ds(base * 4, WIN * 4)], iallv)
        pltpu.sync_copy(wflat.at[pl.ds(base * 4, WIN * 4)], wallv)
        for k in range(3):
            for j in range(WIN // 16):
                ivs[k][pl.ds(j * 16, 16)] = plsc.load_gather(
                    iallv, [(iota16 + 16 * j) * 4 + k])
        c0 = pltpu.async_copy(g2.at[i0v], r0, sem)
        c1 = pltpu.async_copy(g2.at[i1v], r1, sem)
        c2 = pltpu.async_copy(g2.at[i2v], r2, sem)
        c0.wait()
        c1.wait()
        c2.wait()

        @pl.loop(0, WIN)
        def _(i):
            s0 = plsc.load_gather(wallv, [jnp.broadcast_to(4 * i, (16,))])
            s1 = plsc.load_gather(wallv, [jnp.broadcast_to(4 * i + 1, (16,))])
            s2 = plsc.load_gather(wallv, [jnp.broadcast_to(4 * i + 2, (16,))])
            for c in range(O1 // 16):
                cs = pl.ds(c * 16, 16)
                ov[i, cs] = (s0 * r0[i, cs] + s1 * r1[i, cs]
                             + s2 * r2[i, cs])

        pltpu.sync_copy(ov, out.at[pl.ds(base, WIN)])


def _combine_kernel(i_ref, f1_ref, w1a_ref, h1_ref, st_ref):
    h1 = i_ref[0] + jax.lax.dot_general(
        f1_ref[0], w1a_ref[...], (((1,), (1,)), ((), ())),
        preferred_element_type=jnp.float32)
    h1_ref[0] = h1
    part = jnp.concatenate([jnp.sum(h1, axis=0)[None, :],
                            jnp.sum(h1 * h1, axis=0)[None, :]], axis=0)
    first = (pl.program_id(0) == 0) & (pl.program_id(1) == 0)

    @pl.when(first)
    def _():
        st_ref[...] = part

    @pl.when(jnp.logical_not(first))
    def _():
        st_ref[...] += part


def _bn_affine_in_kernel(st_ref, g_ref, b_ref):
    mean = st_ref[0:1, :] * (1.0 / CNT)
    var = jnp.maximum(st_ref[1:2, :] * (1.0 / CNT) - mean * mean, 0.0)
    scale = g_ref[...] * jax.lax.rsqrt(var + EPS_BN)
    shift = b_ref[...] - mean * scale
    return scale, shift


def _layer2_kernel(h1_ref, st1_ref, g_ref, b_ref, w2_ref, h2_ref, st_ref):
    scale, shift = _bn_affine_in_kernel(st1_ref, g_ref, b_ref)
    x = jnp.maximum(h1_ref[0] * scale + shift, 0.0)
    h2 = jax.lax.dot_general(
        x, w2_ref[...], (((1,), (1,)), ((), ())),
        preferred_element_type=jnp.float32)
    h2_ref[0] = h2
    part = jnp.concatenate([jnp.sum(h2, axis=0)[None, :],
                            jnp.sum(h2 * h2, axis=0)[None, :]], axis=0)
    first = (pl.program_id(0) == 0) & (pl.program_id(1) == 0)

    @pl.when(first)
    def _():
        st_ref[...] = part

    @pl.when(jnp.logical_not(first))
    def _():
        st_ref[...] += part


def _final_kernel(h2_ref, st2_ref, g_ref, b_ref, o_ref):
    scale, shift = _bn_affine_in_kernel(st2_ref, g_ref, b_ref)
    o_ref[0] = jnp.maximum(h2_ref[0] * scale + shift, 0.0)


@jax.jit
def _run(point_1, point_2, point_feat_1, point_feat_2, W1, g1, b1, W2, g2, b2):
    W1a = W1[:, :C1]
    W1b = W1[:, C1:]
    g1r = g1.reshape(1, O1)
    b1r = b1.reshape(1, O1)
    g2r = g2.reshape(1, O2)
    b2r = b2.reshape(1, O2)

    G = pl.pallas_call(
        _g_kernel,
        grid=(B,),
        in_specs=[
            pl.BlockSpec((1, S, C2), lambda b: (b, 0, 0)),
            pl.BlockSpec((O1, C2), lambda b: (0, 0)),
        ],
        out_specs=pl.BlockSpec((1, S, O1), lambda b: (b, 0, 0)),
        out_shape=jax.ShapeDtypeStruct((B, S, O1), jnp.float32),
    )(point_feat_2, W1b)

    idx4, w4 = pl.pallas_call(
        _knn_kernel,
        grid=(B, NBLK),
        in_specs=[
            pl.BlockSpec((1, NB, 3), lambda b, i: (b, i, 0)),
            pl.BlockSpec((1, S, 3), lambda b, i: (b, 0, 0)),
        ],
        out_specs=[
            pl.BlockSpec((1, NB, 4), lambda b, i: (b, i, 0)),
            pl.BlockSpec((1, NB, 4), lambda b, i: (b, i, 0)),
        ],
        out_shape=[
            jax.ShapeDtypeStruct((B, N, 4), jnp.int32),
            jax.ShapeDtypeStruct((B, N, 4), jnp.float32),
        ],
    )(point_1, point_2)

    g2f = G.reshape(B * S, O1)

    cp = pltpu.CompilerParams()
    if "needs_layout_passes" in pltpu.CompilerParams.__dataclass_fields__:
        cp = dataclasses.replace(cp, needs_layout_passes=False)
    sc_interp = functools.partial(
        pl.kernel,
        out_type=jax.ShapeDtypeStruct((M, O1), jnp.float32),
        compiler_params=cp,
        mesh=plsc.VectorSubcoreMesh(core_axis_name="c", subcore_axis_name="s"),
        scratch_types=[
            pltpu.VMEM((WIN * 4,), jnp.int32),
            pltpu.VMEM((WIN * 4,), jnp.float32),
            pltpu.VMEM((WIN,), jnp.int32),
            pltpu.VMEM((WIN,), jnp.int32),
            pltpu.VMEM((WIN,), jnp.int32),
            pltpu.VMEM((WIN, O1), jnp.float32),
            pltpu.VMEM((WIN, O1), jnp.float32),
            pltpu.VMEM((WIN, O1), jnp.float32),
            pltpu.VMEM((WIN, O1), jnp.float32),
            pltpu.SemaphoreType.DMA,
        ],
    )(_sc_interp)

    interp = sc_interp(g2f, idx4.reshape(M * 4), w4.reshape(M * 4))
    interp = interp.reshape(B, N, O1)

    h1, st1 = pl.pallas_call(
        _combine_kernel,
        grid=(B, NBLK2),
        in_specs=[
            pl.BlockSpec((1, NB2, O1), lambda b, i: (b, i, 0)),
            pl.BlockSpec((1, NB2, C1), lambda b, i: (b, i, 0)),
            pl.BlockSpec((O1, C1), lambda b, i: (0, 0)),
        ],
        out_specs=[
            pl.BlockSpec((1, NB2, O1), lambda b, i: (b, i, 0)),
            pl.BlockSpec((2, O1), lambda b, i: (0, 0)),
        ],
        out_shape=[
            jax.ShapeDtypeStruct((B, N, O1), jnp.float32),
            jax.ShapeDtypeStruct((2, O1), jnp.float32),
        ],
    )(interp, point_feat_1, W1a)

    h2, st2 = pl.pallas_call(
        _layer2_kernel,
        grid=(B, NBLK2),
        in_specs=[
            pl.BlockSpec((1, NB2, O1), lambda b, i: (b, i, 0)),
            pl.BlockSpec((2, O1), lambda b, i: (0, 0)),
            pl.BlockSpec((1, O1), lambda b, i: (0, 0)),
            pl.BlockSpec((1, O1), lambda b, i: (0, 0)),
            pl.BlockSpec((O2, O1), lambda b, i: (0, 0)),
        ],
        out_specs=[
            pl.BlockSpec((1, NB2, O2), lambda b, i: (b, i, 0)),
            pl.BlockSpec((2, O2), lambda b, i: (0, 0)),
        ],
        out_shape=[
            jax.ShapeDtypeStruct((B, N, O2), jnp.float32),
            jax.ShapeDtypeStruct((2, O2), jnp.float32),
        ],
    )(h1, st1, g1r, b1r, W2)

    out = pl.pallas_call(
        _final_kernel,
        grid=(B, NBLK2),
        in_specs=[
            pl.BlockSpec((1, NB2, O2), lambda b, i: (b, i, 0)),
            pl.BlockSpec((2, O2), lambda b, i: (0, 0)),
            pl.BlockSpec((1, O2), lambda b, i: (0, 0)),
            pl.BlockSpec((1, O2), lambda b, i: (0, 0)),
        ],
        out_specs=pl.BlockSpec((1, NB2, O2), lambda b, i: (b, i, 0)),
        out_shape=jax.ShapeDtypeStruct((B, N, O2), jnp.float32),
    )(h2, st2, g2r, b2r)

    return out


def kernel(point_1, point_2, point_feat_1, point_feat_2, W1, g1, b1, W2, g2, b2):
    return _run(point_1, point_2, point_feat_1, point_feat_2,
                W1, g1, b1, W2, g2, b2)


# speedup vs baseline: 1.6521x; 1.1164x over previous
import dataclasses
import functools

import jax
import jax.numpy as jnp
from jax import lax
from jax.experimental import pallas as pl
from jax.experimental.pallas import tpu as pltpu
from jax.experimental.pallas import tpu_sc as plsc

B, N, S = 8, 4096, 1024
C1, C2 = 256, 512
O1, O2 = 256, 256
EPS_BN = 1e-5
NB = 512
NBLK = N // NB
NB2 = 2048
NBLK2 = N // NB2
CNT = float(B * N)
M = B * N
NWORK = 32
PPW = M // NWORK
WIN = 64
NWIN = PPW // WIN


def _g_kernel(f2_ref, w1b_ref, g_ref):
    g_ref[0] = jax.lax.dot_general(
        f2_ref[0], w1b_ref[...], (((1,), (1,)), ((), ())),
        preferred_element_type=jnp.float32)


def _knn_kernel(p1_ref, p2_ref, idx_ref, w_ref):
    p1 = p1_ref[0]
    p2 = p2_ref[0]
    d = jnp.zeros((NB, S), jnp.float32)
    for j in range(3):
        t = p1[:, j][:, None] - p2[:, j][None, :]
        d = d + t * t
    iota = jax.lax.broadcasted_iota(jnp.int32, (NB, S), 1)
    key = jax.lax.bitcast_convert_type(
        (jax.lax.bitcast_convert_type(d, jnp.int32) & -1024) | iota,
        jnp.float32)
    m1 = jnp.full((NB, 128), 3.0e38, jnp.float32)
    m2 = m1
    m3 = m1
    for c in range(8):
        x = key[:, c * 128:(c + 1) * 128]
        hi = jnp.maximum(m1, x)
        m1 = jnp.minimum(m1, x)
        hi2 = jnp.maximum(m2, hi)
        m2 = jnp.minimum(m2, hi)
        m3 = jnp.minimum(m3, hi2)
    ks = []
    for r in range(3):
        k = jnp.min(m1, axis=1, keepdims=True)
        ks.append(k)
        if r < 2:
            sel = (m1 == k)
            m1 = jnp.where(sel, m2, m1)
            m2 = jnp.where(sel, m3, m2)
    recips = []
    gidxs = []
    boff = pl.program_id(0) * S
    for k in ks:
        ki = jax.lax.bitcast_convert_type(k, jnp.int32)
        dk = jax.lax.bitcast_convert_type(ki & -1024, jnp.float32)
        recips.append(1.0 / (dk + 1e-8))
        gidxs.append((ki & 1023) + boff)
    wsum = recips[0] + recips[1] + recips[2]
    idx_ref[0] = jnp.concatenate(
        gidxs + [jnp.zeros((NB, 1), jnp.int32)], axis=1)
    w_ref[0] = jnp.concatenate(
        [recips[0] / wsum, recips[1] / wsum, recips[2] / wsum,
         jnp.zeros((NB, 1), jnp.float32)], axis=1)


def _sc_interp(g2, iflat, wflat, out,
               ia0, ia1, wa0, wa1, iv0, iv1, r0a, r1a, r2a,
               r0b, r1b, r2b, ov, sema, semb):
    wid = lax.axis_index("s") * 2 + lax.axis_index("c")
    iota16 = jax.lax.iota(jnp.int32, 16)
    rows_a = (r0a, r1a, r2a)
    rows_b = (r0b, r1b, r2b)

    def prefetch(wi, iall, ivbuf, wall, rows, sem):
        base = wid * PPW + wi * WIN
        pltpu.sync_copy(iflat.at[pl.ds(base * 4, WIN * 4)], iall)
        pltpu.sync_copy(wflat.at[pl.ds(base * 4, WIN * 4)], wall)
        for k in range(3):
            for j in range(WIN // 16):
                ivbuf[pl.ds(j * 16, 16)] = plsc.load_gather(
                    iall, [(iota16 + 16 * j) * 4 + k])
            pltpu.async_copy(g2.at[ivbuf], rows[k], sem)

    def compute(wi, wall, ivbuf, rows, sem):
        for k in range(3):
            pltpu.make_async_copy(g2.at[ivbuf], rows[k], sem).wait()

        @pl.loop(0, WIN)
        def _(i):
            s0 = plsc.load_gather(wall, [jnp.broadcast_to(4 * i, (16,))])
            s1 = plsc.load_gather(wall, [jnp.broadcast_to(4 * i + 1, (16,))])
            s2 = plsc.load_gather(wall, [jnp.broadcast_to(4 * i + 2, (16,))])
            for c in range(O1 // 16):
                cs = pl.ds(c * 16, 16)
                ov[i, cs] = (s0 * rows[0][i, cs] + s1 * rows[1][i, cs]
                             + s2 * rows[2][i, cs])

        base = wid * PPW + wi * WIN
        pltpu.sync_copy(ov, out.at[pl.ds(base, WIN)])

    prefetch(0, ia0, iv0, wa0, rows_a, sema)

    @pl.loop(0, NWIN // 2)
    def _(t):
        prefetch(2 * t + 1, ia1, iv1, wa1, rows_b, semb)
        compute(2 * t, wa0, iv0, rows_a, sema)

        @pl.when(t < NWIN // 2 - 1)
        def _():
            prefetch(2 * t + 2, ia0, iv0, wa0, rows_a, sema)

        compute(2 * t + 1, wa1, iv1, rows_b, semb)


def _combine_kernel(i_ref, f1_ref, w1a_ref, h1_ref, st_ref):
    h1 = i_ref[0] + jax.lax.dot_general(
        f1_ref[0], w1a_ref[...], (((1,), (1,)), ((), ())),
        preferred_element_type=jnp.float32)
    h1_ref[0] = h1
    part = jnp.concatenate([jnp.sum(h1, axis=0)[None, :],
                            jnp.sum(h1 * h1, axis=0)[None, :]], axis=0)
    first = (pl.program_id(0) == 0) & (pl.program_id(1) == 0)

    @pl.when(first)
    def _():
        st_ref[...] = part

    @pl.when(jnp.logical_not(first))
    def _():
        st_ref[...] += part


def _bn_affine_in_kernel(st_ref, g_ref, b_ref):
    mean = st_ref[0:1, :] * (1.0 / CNT)
    var = jnp.maximum(st_ref[1:2, :] * (1.0 / CNT) - mean * mean, 0.0)
    scale = g_ref[...] * jax.lax.rsqrt(var + EPS_BN)
    shift = b_ref[...] - mean * scale
    return scale, shift


def _layer2_kernel(h1_ref, st1_ref, g_ref, b_ref, w2_ref, h2_ref, st_ref):
    scale, shift = _bn_affine_in_kernel(st1_ref, g_ref, b_ref)
    x = jnp.maximum(h1_ref[0] * scale + shift, 0.0)
    h2 = jax.lax.dot_general(
        x, w2_ref[...], (((1,), (1,)), ((), ())),
        preferred_element_type=jnp.float32)
    h2_ref[0] = h2
    part = jnp.concatenate([jnp.sum(h2, axis=0)[None, :],
                            jnp.sum(h2 * h2, axis=0)[None, :]], axis=0)
    first = (pl.program_id(0) == 0) & (pl.program_id(1) == 0)

    @pl.when(first)
    def _():
        st_ref[...] = part

    @pl.when(jnp.logical_not(first))
    def _():
        st_ref[...] += part


def _final_kernel(h2_ref, st2_ref, g_ref, b_ref, o_ref):
    scale, shift = _bn_affine_in_kernel(st2_ref, g_ref, b_ref)
    o_ref[0] = jnp.maximum(h2_ref[0] * scale + shift, 0.0)


@jax.jit
def _run(point_1, point_2, point_feat_1, point_feat_2, W1, g1, b1, W2, g2, b2):
    W1a = W1[:, :C1]
    W1b = W1[:, C1:]
    g1r = g1.reshape(1, O1)
    b1r = b1.reshape(1, O1)
    g2r = g2.reshape(1, O2)
    b2r = b2.reshape(1, O2)

    G = pl.pallas_call(
        _g_kernel,
        grid=(B,),
        in_specs=[
            pl.BlockSpec((1, S, C2), lambda b: (b, 0, 0)),
            pl.BlockSpec((O1, C2), lambda b: (0, 0)),
        ],
        out_specs=pl.BlockSpec((1, S, O1), lambda b: (b, 0, 0)),
        out_shape=jax.ShapeDtypeStruct((B, S, O1), jnp.float32),
    )(point_feat_2, W1b)

    idx4, w4 = pl.pallas_call(
        _knn_kernel,
        grid=(B, NBLK),
        in_specs=[
            pl.BlockSpec((1, NB, 3), lambda b, i: (b, i, 0)),
            pl.BlockSpec((1, S, 3), lambda b, i: (b, 0, 0)),
        ],
        out_specs=[
            pl.BlockSpec((1, NB, 4), lambda b, i: (b, i, 0)),
            pl.BlockSpec((1, NB, 4), lambda b, i: (b, i, 0)),
        ],
        out_shape=[
            jax.ShapeDtypeStruct((B, N, 4), jnp.int32),
            jax.ShapeDtypeStruct((B, N, 4), jnp.float32),
        ],
    )(point_1, point_2)

    g2f = G.reshape(B * S, O1)

    cp = pltpu.CompilerParams()
    if "needs_layout_passes" in pltpu.CompilerParams.__dataclass_fields__:
        cp = dataclasses.replace(cp, needs_layout_passes=False)
    sc_interp = functools.partial(
        pl.kernel,
        out_type=jax.ShapeDtypeStruct((M, O1), jnp.float32),
        compiler_params=cp,
        mesh=plsc.VectorSubcoreMesh(core_axis_name="c", subcore_axis_name="s"),
        scratch_types=[
            pltpu.VMEM((WIN * 4,), jnp.int32),
            pltpu.VMEM((WIN * 4,), jnp.int32),
            pltpu.VMEM((WIN * 4,), jnp.float32),
            pltpu.VMEM((WIN * 4,), jnp.float32),
            pltpu.VMEM((WIN,), jnp.int32),
            pltpu.VMEM((WIN,), jnp.int32),
            pltpu.VMEM((WIN, O1), jnp.float32),
            pltpu.VMEM((WIN, O1), jnp.float32),
            pltpu.VMEM((WIN, O1), jnp.float32),
            pltpu.VMEM((WIN, O1), jnp.float32),
            pltpu.VMEM((WIN, O1), jnp.float32),
            pltpu.VMEM((WIN, O1), jnp.float32),
            pltpu.VMEM((WIN, O1), jnp.float32),
            pltpu.SemaphoreType.DMA,
            pltpu.SemaphoreType.DMA,
        ],
    )(_sc_interp)

    interp = sc_interp(g2f, idx4.reshape(M * 4), w4.reshape(M * 4))
    interp = interp.reshape(B, N, O1)

    h1, st1 = pl.pallas_call(
        _combine_kernel,
        grid=(B, NBLK2),
        in_specs=[
            pl.BlockSpec((1, NB2, O1), lambda b, i: (b, i, 0)),
            pl.BlockSpec((1, NB2, C1), lambda b, i: (b, i, 0)),
            pl.BlockSpec((O1, C1), lambda b, i: (0, 0)),
        ],
        out_specs=[
            pl.BlockSpec((1, NB2, O1), lambda b, i: (b, i, 0)),
            pl.BlockSpec((2, O1), lambda b, i: (0, 0)),
        ],
        out_shape=[
            jax.ShapeDtypeStruct((B, N, O1), jnp.float32),
            jax.ShapeDtypeStruct((2, O1), jnp.float32),
        ],
    )(interp, point_feat_1, W1a)

    h2, st2 = pl.pallas_call(
        _layer2_kernel,
        grid=(B, NBLK2),
        in_specs=[
            pl.BlockSpec((1, NB2, O1), lambda b, i: (b, i, 0)),
            pl.BlockSpec((2, O1), lambda b, i: (0, 0)),
            pl.BlockSpec((1, O1), lambda b, i: (0, 0)),
            pl.BlockSpec((1, O1), lambda b, i: (0, 0)),
            pl.BlockSpec((O2, O1), lambda b, i: (0, 0)),
        ],
        out_specs=[
            pl.BlockSpec((1, NB2, O2), lambda b, i: (b, i, 0)),
            pl.BlockSpec((2, O2), lambda b, i: (0, 0)),
        ],
        out_shape=[
            jax.ShapeDtypeStruct((B, N, O2), jnp.float32),
            jax.ShapeDtypeStruct((2, O2), jnp.float32),
        ],
    )(h1, st1, g1r, b1r, W2)

    out = pl.pallas_call(
        _final_kernel,
        grid=(B, NBLK2),
        in_specs=[
            pl.BlockSpec((1, NB2, O2), lambda b, i: (b, i, 0)),
            pl.BlockSpec((2, O2), lambda b, i: (0, 0)),
            pl.BlockSpec((1, O2), lambda b, i: (0, 0)),
            pl.BlockSpec((1, O2), lambda b, i: (0, 0)),
        ],
        out_specs=pl.BlockSpec((1, NB2, O2), lambda b, i: (b, i, 0)),
        out_shape=jax.ShapeDtypeStruct((B, N, O2), jnp.float32),
    )(h2, st2, g2r, b2r)

    return out


def kernel(point_1, point_2, point_feat_1, point_feat_2, W1, g1, b1, W2, g2, b2):
    return _run(point_1, point_2, point_feat_1, point_feat_2,
                W1, g1, b1, W2, g2, b2)
